# baseline (speedup 1.0000x reference)
import functools

import jax
import jax.numpy as jnp
from jax import lax
from jax.experimental import pallas as pl
from jax.experimental.pallas import tpu as pltpu
from jax.experimental.pallas import tpu_sc as plsc

_DIM = 64
_LANES = 128
_N_WORKERS = 32
_TC_COLS = 1024


def _iota16():
    return lax.iota(jnp.int32, 16)


def _tc_transpose_kernel(x_ref, o_ref):
    x = x_ref[...]
    o_ref[...] = x.reshape(_DIM, _TC_COLS // 2, 2).transpose(1, 2, 0).reshape(
        _TC_COLS // 2, 2 * _DIM)


def _gather_kernel(ids_hbm, pair_hbm, out_hbm, idx_v, idxp_v,
                   gbuf0, gbuf1, obuf0, obuf1,
                   gsem0, gsem1, wsem0, wsem1, *, rows_per_w):
    wid = lax.axis_index("s") * 2 + lax.axis_index("c")
    gbufs = (gbuf0, gbuf1)
    obufs = (obuf0, obuf1)
    gsems = (gsem0, gsem1)
    wsems = (wsem0, wsem1)
    iot = _iota16()
    bvecs = [iot + 16 * q for q in range(8)]

    pltpu.sync_copy(ids_hbm.at[:, pl.ds(wid * _LANES, _LANES)], idx_v)

    def idx_body(j, carry):
        for g in range(_LANES // 16):
            sl = pl.ds(g * 16, 16)
            idxp_v[j, sl] = lax.shift_right_logical(idx_v[j, sl], 1)
        return carry
    lax.fori_loop(0, rows_per_w, idx_body, 0)

    def gather_start(s, u):
        pltpu.make_async_copy(pair_hbm.at[idxp_v.at[s]], gbufs[u],
                              gsems[u]).start()

    def gather_wait(u):
        pltpu.make_async_copy(pair_hbm.at[idxp_v.at[0]], gbufs[u],
                              gsems[u]).wait()

    def wb_start(s, u):
        pltpu.make_async_copy(
            obufs[u], out_hbm.at[s, :, pl.ds(wid * _LANES, _LANES)],
            wsems[u]).start()

    def wb_wait(u):
        pltpu.make_async_copy(
            obufs[u], out_hbm.at[0, :, pl.ds(wid * _LANES, _LANES)],
            wsems[u]).wait()

    def merge(s, u):
        g, o = gbufs[u], obufs[u]
        for q in range(8):
            hv = (idx_v[s, pl.ds(16 * q, 16)] & 1) * _DIM

            @plsc.parallel_loop(0, _DIM, unroll=8)
            def _(dd):
                o[dd, pl.ds(16 * q, 16)] = plsc.load_gather(
                    g, [bvecs[q], hv + dd])

    gather_start(0, 0)
    gather_start(1, 1)

    def block(it, carry):
        for u in range(2):
            s = 2 * it + u
            gather_wait(u)

            @pl.when(s >= 2)
            def _():
                wb_wait(u)
            merge(s, u)
            wb_start(s, u)

            @pl.when(s + 2 < rows_per_w)
            def _():
                gather_start(s + 2, u)
        return carry

    lax.fori_loop(0, rows_per_w // 2, block, 0)
    wb_wait(0)
    wb_wait(1)


def kernel(input_ids, weight):
    b, s = input_ids.shape
    n, d = weight.shape
    mesh = plsc.VectorSubcoreMesh(core_axis_name="c", subcore_axis_name="s")
    cparams = pltpu.CompilerParams(use_tc_tiling_on_sc=True,
                                   needs_layout_passes=False)

    wt = weight.T
    ids_t = input_ids.T.astype(jnp.int32)

    n_blocks = (n + _TC_COLS - 1) // _TC_COLS
    pair = pl.pallas_call(
        _tc_transpose_kernel,
        grid=(n_blocks,),
        in_specs=[pl.BlockSpec((d, _TC_COLS), lambda i: (0, i))],
        out_specs=pl.BlockSpec((_TC_COLS // 2, 2 * d), lambda i: (i, 0)),
        out_shape=jax.ShapeDtypeStruct((n // 2, 2 * d), jnp.float32),
    )(wt)

    gather_run = functools.partial(
        pl.kernel,
        mesh=mesh,
        out_type=jax.ShapeDtypeStruct((s, d, b), jnp.float32),
        scratch_types=[
            pltpu.VMEM((s, _LANES), jnp.int32),
            pltpu.VMEM((s, _LANES), jnp.int32),
            pltpu.VMEM((_LANES, 2 * d), jnp.float32),
            pltpu.VMEM((_LANES, 2 * d), jnp.float32),
            pltpu.VMEM((d, _LANES), jnp.float32),
            pltpu.VMEM((d, _LANES), jnp.float32),
            pltpu.SemaphoreType.DMA,
            pltpu.SemaphoreType.DMA,
            pltpu.SemaphoreType.DMA,
            pltpu.SemaphoreType.DMA,
        ],
        compiler_params=cparams,
    )(functools.partial(_gather_kernel, rows_per_w=s))

    out_t = gather_run(ids_t, pair)
    return out_t.transpose(2, 0, 1)
